# Initial kernel scaffold; baseline (speedup 1.0000x reference)
import jax
import jax.numpy as jnp
from jax.experimental import pallas as pl


def kernel(images, features, conv_w, conv_b, cls_w, cls_b, bbox_w, bbox_b):
    raise NotImplementedError("write your pallas kernel here")


# speedup vs baseline: 28.6824x; 28.6824x over previous
import functools

import jax
import jax.numpy as jnp
import numpy as np
from jax import lax
from jax.experimental import pallas as pl
from jax.experimental.pallas import tpu as pltpu

STRIDE = 16
SIZES = (32.0, 64.0, 128.0, 256.0, 512.0)
PRE_NMS_TOP_N = 2000
POST_NMS_TOP_N = 1000
NMS_THRESH = 0.7
BBOX_XFORM_CLIP = float(np.log(1000.0 / 16.0))
KPAD = 2048


def _nms_fixpoint_kernel(boxes_ref, keep_ref):
    b = boxes_ref[:]
    x1 = b[:, 0:1]
    y1 = b[:, 1:2]
    x2 = b[:, 2:3]
    y2 = b[:, 3:4]
    area = (x2 - x1 + 1.0) * (y2 - y1 + 1.0)

    x1r = jnp.transpose(x1)
    y1r = jnp.transpose(y1)
    x2r = jnp.transpose(x2)
    y2r = jnp.transpose(y2)
    arear = jnp.transpose(area)

    lt_x = jnp.maximum(x1, x1r)
    lt_y = jnp.maximum(y1, y1r)
    rb_x = jnp.minimum(x2, x2r)
    rb_y = jnp.minimum(y2, y2r)
    w = jnp.maximum(rb_x - lt_x + 1.0, 0.0)
    h = jnp.maximum(rb_y - lt_y + 1.0, 0.0)
    inter = w * h
    iou = inter / (area + arear - inter)

    jj = lax.broadcasted_iota(jnp.int32, (KPAD, KPAD), 0)
    ii = lax.broadcasted_iota(jnp.int32, (KPAD, KPAD), 1)
    valid = (jj < ii) & (ii < PRE_NMS_TOP_N) & (jj < PRE_NMS_TOP_N)
    m = jnp.where((iou > NMS_THRESH) & valid, 1.0, 0.0)

    keep0 = jnp.ones((8, KPAD), dtype=jnp.float32)

    def body(carry):
        keep, _ = carry
        s = jnp.dot(keep, m, preferred_element_type=jnp.float32)
        new = jnp.where(s == 0.0, 1.0, 0.0)
        changed = jnp.sum(jnp.abs(new - keep)) > 0.0
        return new, changed

    def cond(carry):
        return carry[1]

    keep, _ = lax.while_loop(cond, body, (keep0, jnp.bool_(True)))
    keep_ref[:] = keep[0:1, :]


def _nms_keep_pallas(boxes):
    bp = jnp.zeros((KPAD, 4), dtype=jnp.float32).at[:PRE_NMS_TOP_N].set(boxes)
    keep = pl.pallas_call(
        _nms_fixpoint_kernel,
        out_shape=jax.ShapeDtypeStruct((1, KPAD), jnp.float32),
    )(bp)
    return keep[0, :PRE_NMS_TOP_N] > 0.5


def _make_anchors(H, W):
    sizes = np.array(SIZES, dtype=np.float64)
    cell = np.stack([-(sizes - 1) / 2.0, -(sizes - 1) / 2.0,
                     (sizes - 1) / 2.0, (sizes - 1) / 2.0], axis=1)
    shift_x = np.arange(W, dtype=np.float64) * STRIDE
    shift_y = np.arange(H, dtype=np.float64) * STRIDE
    sy, sx = np.meshgrid(shift_y, shift_x, indexing="ij")
    shifts = np.stack([sx.ravel(), sy.ravel(), sx.ravel(), sy.ravel()], axis=1)
    anchors = (shifts[:, None, :] + cell[None, :, :]).reshape(-1, 4)
    return jnp.asarray(anchors, dtype=jnp.float32)


def _decode(deltas, anchors):
    w = anchors[:, 2] - anchors[:, 0] + 1.0
    h = anchors[:, 3] - anchors[:, 1] + 1.0
    cx = anchors[:, 0] + 0.5 * w
    cy = anchors[:, 1] + 0.5 * h
    dx, dy = deltas[:, 0], deltas[:, 1]
    dw = jnp.minimum(deltas[:, 2], BBOX_XFORM_CLIP)
    dh = jnp.minimum(deltas[:, 3], BBOX_XFORM_CLIP)
    pcx = dx * w + cx
    pcy = dy * h + cy
    pw = jnp.exp(dw) * w
    ph = jnp.exp(dh) * h
    x1 = pcx - 0.5 * pw
    y1 = pcy - 0.5 * ph
    x2 = pcx + 0.5 * pw - 1.0
    y2 = pcy + 0.5 * ph - 1.0
    return jnp.stack([x1, y1, x2, y2], axis=1)


def kernel(images, features, conv_w, conv_b, cls_w, cls_b, bbox_w, bbox_b):
    dn = ("NCHW", "OIHW", "NCHW")
    t = lax.conv_general_dilated(features, conv_w, (1, 1), "SAME",
                                 dimension_numbers=dn)
    t = jax.nn.relu(t + conv_b[None, :, None, None])
    obj = lax.conv_general_dilated(t, cls_w, (1, 1), "SAME",
                                   dimension_numbers=dn) + cls_b[None, :, None, None]
    reg = lax.conv_general_dilated(t, bbox_w, (1, 1), "SAME",
                                   dimension_numbers=dn) + bbox_b[None, :, None, None]
    A = cls_w.shape[0]
    H, W = features.shape[2], features.shape[3]
    obj = jnp.transpose(obj[0], (1, 2, 0)).reshape(-1)
    reg = jnp.transpose(reg[0].reshape(A, 4, H, W), (2, 3, 0, 1)).reshape(-1, 4)
    anchors = _make_anchors(H, W)
    scores = jax.nn.sigmoid(obj)
    K = PRE_NMS_TOP_N
    top_scores, top_idx = lax.top_k(scores, K)
    boxes = _decode(reg[top_idx], anchors[top_idx])
    im_h = float(images.shape[2]); im_w = float(images.shape[3])
    boxes = jnp.stack([
        jnp.clip(boxes[:, 0], 0.0, im_w - 1.0),
        jnp.clip(boxes[:, 1], 0.0, im_h - 1.0),
        jnp.clip(boxes[:, 2], 0.0, im_w - 1.0),
        jnp.clip(boxes[:, 3], 0.0, im_h - 1.0),
    ], axis=1)
    keep = _nms_keep_pallas(boxes)
    masked = jnp.where(keep, top_scores, -1.0)
    _, final_idx = lax.top_k(masked, POST_NMS_TOP_N)
    out_boxes = boxes[final_idx]
    out_scores = top_scores[final_idx]
    return jnp.concatenate([out_boxes, out_scores[:, None]], axis=1)


# speedup vs baseline: 35.4078x; 1.2345x over previous
import functools

import jax
import jax.numpy as jnp
import numpy as np
from jax import lax
from jax.experimental import pallas as pl
from jax.experimental.pallas import tpu as pltpu

STRIDE = 16
SIZES = (32.0, 64.0, 128.0, 256.0, 512.0)
PRE_NMS_TOP_N = 2000
POST_NMS_TOP_N = 1000
NMS_THRESH = 0.7
BBOX_XFORM_CLIP = float(np.log(1000.0 / 16.0))
KPAD = 2048


def _nms_fixpoint_kernel(boxes_ref, keep_ref):
    b = boxes_ref[:]
    x1 = b[:, 0:1]
    y1 = b[:, 1:2]
    x2 = b[:, 2:3]
    y2 = b[:, 3:4]
    area = (x2 - x1 + 1.0) * (y2 - y1 + 1.0)

    x1r = jnp.transpose(x1)
    y1r = jnp.transpose(y1)
    x2r = jnp.transpose(x2)
    y2r = jnp.transpose(y2)
    arear = jnp.transpose(area)

    lt_x = jnp.maximum(x1, x1r)
    lt_y = jnp.maximum(y1, y1r)
    rb_x = jnp.minimum(x2, x2r)
    rb_y = jnp.minimum(y2, y2r)
    w = jnp.maximum(rb_x - lt_x + 1.0, 0.0)
    h = jnp.maximum(rb_y - lt_y + 1.0, 0.0)
    inter = w * h
    iou = inter / (area + arear - inter)

    jj = lax.broadcasted_iota(jnp.int32, (KPAD, KPAD), 0)
    ii = lax.broadcasted_iota(jnp.int32, (KPAD, KPAD), 1)
    valid = (jj < ii) & (ii < PRE_NMS_TOP_N) & (jj < PRE_NMS_TOP_N)
    m = jnp.where((iou > NMS_THRESH) & valid, 1.0, 0.0)

    keep0 = jnp.ones((8, KPAD), dtype=jnp.float32)

    def body(carry):
        keep, _ = carry
        s = jnp.dot(keep, m, preferred_element_type=jnp.float32)
        new = jnp.where(s == 0.0, 1.0, 0.0)
        changed = jnp.sum(jnp.abs(new - keep)) > 0.0
        return new, changed

    def cond(carry):
        return carry[1]

    keep, _ = lax.while_loop(cond, body, (keep0, jnp.bool_(True)))
    keep_ref[:] = keep[0:1, :]


def _nms_keep_pallas(boxes):
    bp = jnp.zeros((KPAD, 4), dtype=jnp.float32).at[:PRE_NMS_TOP_N].set(boxes)
    keep = pl.pallas_call(
        _nms_fixpoint_kernel,
        out_shape=jax.ShapeDtypeStruct((1, KPAD), jnp.float32),
    )(bp)
    return keep[0, :PRE_NMS_TOP_N] > 0.5


def _conv_head_kernel(f_ref, w9_ref, cb_ref, hw_ref, hb_ref, out_ref):
    acc = jnp.zeros((4224, 256), dtype=jnp.float32)
    for t in range(9):
        off = (t // 3) * 66 + (t % 3)
        acc = acc + jnp.dot(f_ref[off:off + 4224, :], w9_ref[t],
                            preferred_element_type=jnp.float32)
    act = jax.nn.relu(acc + cb_ref[0][None, :])
    out_ref[:] = jnp.dot(act, hw_ref[:],
                         preferred_element_type=jnp.float32) + hb_ref[0][None, :]


def _conv_head_pallas(features, conv_w, conv_b, cls_w, cls_b, bbox_w, bbox_b):
    feat = jnp.transpose(features[0], (1, 2, 0))
    fpad = jnp.pad(feat, ((1, 1), (1, 1), (0, 0))).reshape(4356, 256)
    fpad = jnp.pad(fpad, ((0, 4), (0, 0)))
    w9 = jnp.transpose(conv_w, (2, 3, 1, 0)).reshape(9, 256, 256)
    hw = jnp.zeros((256, 128), jnp.float32)
    hw = hw.at[:, 0:5].set(jnp.transpose(cls_w[:, :, 0, 0]))
    hw = hw.at[:, 8:28].set(jnp.transpose(bbox_w[:, :, 0, 0]))
    hb = jnp.zeros((1, 128), jnp.float32)
    hb = hb.at[0, 0:5].set(cls_b)
    hb = hb.at[0, 8:28].set(bbox_b)
    return pl.pallas_call(
        _conv_head_kernel,
        out_shape=jax.ShapeDtypeStruct((4224, 128), jnp.float32),
    )(fpad, w9, conv_b.reshape(1, 256), hw, hb)


def _make_anchors(H, W):
    sizes = np.array(SIZES, dtype=np.float64)
    cell = np.stack([-(sizes - 1) / 2.0, -(sizes - 1) / 2.0,
                     (sizes - 1) / 2.0, (sizes - 1) / 2.0], axis=1)
    shift_x = np.arange(W, dtype=np.float64) * STRIDE
    shift_y = np.arange(H, dtype=np.float64) * STRIDE
    sy, sx = np.meshgrid(shift_y, shift_x, indexing="ij")
    shifts = np.stack([sx.ravel(), sy.ravel(), sx.ravel(), sy.ravel()], axis=1)
    anchors = (shifts[:, None, :] + cell[None, :, :]).reshape(-1, 4)
    return jnp.asarray(anchors, dtype=jnp.float32)


def _decode(deltas, anchors):
    w = anchors[:, 2] - anchors[:, 0] + 1.0
    h = anchors[:, 3] - anchors[:, 1] + 1.0
    cx = anchors[:, 0] + 0.5 * w
    cy = anchors[:, 1] + 0.5 * h
    dx, dy = deltas[:, 0], deltas[:, 1]
    dw = jnp.minimum(deltas[:, 2], BBOX_XFORM_CLIP)
    dh = jnp.minimum(deltas[:, 3], BBOX_XFORM_CLIP)
    pcx = dx * w + cx
    pcy = dy * h + cy
    pw = jnp.exp(dw) * w
    ph = jnp.exp(dh) * h
    x1 = pcx - 0.5 * pw
    y1 = pcy - 0.5 * ph
    x2 = pcx + 0.5 * pw - 1.0
    y2 = pcy + 0.5 * ph - 1.0
    return jnp.stack([x1, y1, x2, y2], axis=1)


def kernel(images, features, conv_w, conv_b, cls_w, cls_b, bbox_w, bbox_b):
    out = _conv_head_pallas(features, conv_w, conv_b, cls_w, cls_b,
                            bbox_w, bbox_b)
    obj = out[:, 0:5].reshape(-1)
    reg = out[:, 8:28].reshape(4224, 5, 4).reshape(-1, 4)
    anchors = _make_anchors(64, 66)
    valid = (jnp.arange(21120) // 5) % 66 < 64
    scores = jnp.where(valid, jax.nn.sigmoid(obj), -1.0)
    K = PRE_NMS_TOP_N
    top_scores, top_idx = lax.top_k(scores, K)
    boxes = _decode(reg[top_idx], anchors[top_idx])
    im_h = float(images.shape[2]); im_w = float(images.shape[3])
    boxes = jnp.stack([
        jnp.clip(boxes[:, 0], 0.0, im_w - 1.0),
        jnp.clip(boxes[:, 1], 0.0, im_h - 1.0),
        jnp.clip(boxes[:, 2], 0.0, im_w - 1.0),
        jnp.clip(boxes[:, 3], 0.0, im_h - 1.0),
    ], axis=1)
    keep = _nms_keep_pallas(boxes)
    masked = jnp.where(keep, top_scores, -1.0)
    _, final_idx = lax.top_k(masked, POST_NMS_TOP_N)
    out_boxes = boxes[final_idx]
    out_scores = top_scores[final_idx]
    return jnp.concatenate([out_boxes, out_scores[:, None]], axis=1)
